# Initial kernel scaffold; baseline (speedup 1.0000x reference)
import jax
import jax.numpy as jnp
from jax.experimental import pallas as pl


def kernel(user_ids, movie_ids_1, movie_ids_2, user_table, movie_table, W1, b1, W2, b2):
    raise NotImplementedError("write your pallas kernel here")


# speedup vs baseline: 1.1461x; 1.1461x over previous
import functools

import jax
import jax.numpy as jnp
from jax import lax
from jax.experimental import pallas as pl
from jax.experimental.pallas import tpu as pltpu
from jax.experimental.pallas import tpu_sc as plsc

BATCH = 16384
EMBED_DIM = 32
HIDDEN_DIM = 64
CHUNK = 128

_info = plsc.get_sparse_core_info()
NC, NS = _info.num_cores, _info.num_subcores
NW = NC * NS
B_PER_W = BATCH // NW
NCH = B_PER_W // CHUNK


def _sc_gather(user_table, movie_table, uidx, m1idx, m2idx):
    mesh = plsc.VectorSubcoreMesh(core_axis_name="c", subcore_axis_name="s")
    out_t = jax.ShapeDtypeStruct((BATCH, EMBED_DIM), jnp.float32)

    @functools.partial(
        pl.kernel,
        mesh=mesh,
        out_type=[out_t, out_t, out_t],
        compiler_params=pltpu.CompilerParams(use_tc_tiling_on_sc=False),
        scratch_types=[
            pltpu.VMEM((NCH, CHUNK), jnp.int32),
            pltpu.VMEM((NCH, CHUNK), jnp.int32),
            pltpu.VMEM((NCH, CHUNK), jnp.int32),
            pltpu.VMEM((B_PER_W, EMBED_DIM), jnp.float32),
            pltpu.VMEM((B_PER_W, EMBED_DIM), jnp.float32),
            pltpu.VMEM((B_PER_W, EMBED_DIM), jnp.float32),
            pltpu.SemaphoreType.DMA,
        ],
    )
    def k(ut_hbm, mt_hbm, ui_hbm, m1i_hbm, m2i_hbm,
          u_out, m1_out, m2_out,
          ui_v, m1i_v, m2i_v, ur_v, m1r_v, m2r_v, sem):
        wid = lax.axis_index("s") * NC + lax.axis_index("c")
        base = wid * B_PER_W
        pltpu.sync_copy(ui_hbm.at[wid], ui_v)
        pltpu.sync_copy(m1i_hbm.at[wid], m1i_v)
        pltpu.sync_copy(m2i_hbm.at[wid], m2i_v)
        cps = []
        for ch in range(NCH):
            sl = pl.ds(ch * CHUNK, CHUNK)
            cps.append(pltpu.async_copy(ut_hbm.at[ui_v.at[ch]], ur_v.at[sl], sem))
            cps.append(pltpu.async_copy(mt_hbm.at[m1i_v.at[ch]], m1r_v.at[sl], sem))
            cps.append(pltpu.async_copy(mt_hbm.at[m2i_v.at[ch]], m2r_v.at[sl], sem))
        for cp in cps:
            cp.wait()
        osl = pl.ds(base, B_PER_W)
        pltpu.sync_copy(ur_v, u_out.at[osl])
        pltpu.sync_copy(m1r_v, m1_out.at[osl])
        pltpu.sync_copy(m2r_v, m2_out.at[osl])

    return k(user_table, movie_table, uidx, m1idx, m2idx)


_BLK = 2048


def _mlp_body(u_ref, m1_ref, m2_ref, w1u_ref, w1m_ref, b1_ref, w2_ref, out_ref):
    w1u = w1u_ref[...]
    w1m = w1m_ref[...]
    U = jnp.dot(u_ref[...], w1u, preferred_element_type=jnp.float32)
    M1 = jnp.dot(m1_ref[...], w1m, preferred_element_type=jnp.float32)
    M2 = jnp.dot(m2_ref[...], w1m, preferred_element_type=jnp.float32)
    b1r = b1_ref[...]
    h1 = jnp.maximum(U + M1 + b1r, 0.0)
    h2 = jnp.maximum(U + M2 + b1r, 0.0)
    out_ref[...] = jnp.sum((h1 - h2) * w2_ref[...], axis=1, keepdims=True)


def _tc_mlp(u_emb, m1_emb, m2_emb, W1, b1, W2):
    w1u = W1[:EMBED_DIM]
    w1m = W1[EMBED_DIM:]
    b1r = b1.reshape(1, HIDDEN_DIM)
    w2r = W2.reshape(1, HIDDEN_DIM)
    grid = (BATCH // _BLK,)
    return pl.pallas_call(
        _mlp_body,
        grid=grid,
        in_specs=[
            pl.BlockSpec((_BLK, EMBED_DIM), lambda i: (i, 0)),
            pl.BlockSpec((_BLK, EMBED_DIM), lambda i: (i, 0)),
            pl.BlockSpec((_BLK, EMBED_DIM), lambda i: (i, 0)),
            pl.BlockSpec((EMBED_DIM, HIDDEN_DIM), lambda i: (0, 0)),
            pl.BlockSpec((EMBED_DIM, HIDDEN_DIM), lambda i: (0, 0)),
            pl.BlockSpec((1, HIDDEN_DIM), lambda i: (0, 0)),
            pl.BlockSpec((1, HIDDEN_DIM), lambda i: (0, 0)),
        ],
        out_specs=pl.BlockSpec((_BLK, 1), lambda i: (i, 0)),
        out_shape=jax.ShapeDtypeStruct((BATCH, 1), jnp.float32),
    )(u_emb, m1_emb, m2_emb, w1u, w1m, b1r, w2r)


def kernel(user_ids, movie_ids_1, movie_ids_2, user_table, movie_table,
           W1, b1, W2, b2):
    uidx = user_ids.astype(jnp.int32).reshape(NW, NCH, CHUNK)
    m1idx = movie_ids_1.astype(jnp.int32).reshape(NW, NCH, CHUNK)
    m2idx = movie_ids_2.astype(jnp.int32).reshape(NW, NCH, CHUNK)
    u_emb, m1_emb, m2_emb = _sc_gather(user_table, movie_table,
                                       uidx, m1idx, m2idx)
    return _tc_mlp(u_emb, m1_emb, m2_emb, W1, b1, W2)
